# baseline (speedup 1.0000x reference)
import jax
import jax.numpy as jnp
from jax import lax
from jax.experimental import pallas as pl
from jax.experimental.pallas import tpu as pltpu
from jax.experimental.pallas import tpu_sc as plsc

DELTA_VAR = 0.5
DELTA_DIST = 1.5
ALPHA = 1.0
BETA = 1.0

L = 16
NC = 2
NS = 16
D = 16
C = 128
ROWS = 144


def _rsqrt(x):
    i = plsc.bitcast(x, jnp.int32)
    i = 0x5F3759DF - (i >> 1)
    y = plsc.bitcast(i, jnp.float32)
    for _ in range(3):
        y = y * (1.5 - 0.5 * x * y * y)
    return y


def _tree_sum(xs):
    xs = list(xs)
    while len(xs) > 1:
        nxt = [xs[i] + xs[i + 1] for i in range(0, len(xs) - 1, 2)]
        if len(xs) % 2:
            nxt.append(xs[-1])
        xs = nxt
    return xs[0]


def _body(emb_hbm, seg_hbm, pack_hbm, out_hbm,
          emb_v, seg_v, tab_v, e0_v, e1_v, w_v, idxa_v, idxb_v,
          row_v, shared, sem_in):
    cid = lax.axis_index("c")
    sid = lax.axis_index("s")
    wid = cid * NS + sid

    pix = emb_v.shape[1] * emb_v.shape[2]
    ngrp = pix // L
    ept = e0_v.shape[0]

    iota = lax.iota(jnp.int32, L)
    zeros = jnp.zeros((L,), jnp.float32)
    ones = jnp.ones((L,), jnp.float32)
    cols = [jnp.full((L,), d, jnp.int32) for d in range(D)]

    rpt = pix // 128
    n_edges = ept * NS
    dins = [
        pltpu.make_async_copy(emb_hbm.at[cid, :, pl.ds(sid * rpt, rpt), :],
                              emb_v, sem_in),
        pltpu.make_async_copy(
            seg_hbm.at[pl.ds(cid * NS * pix + sid * pix, pix)], seg_v,
            sem_in),
        pltpu.make_async_copy(pack_hbm.at[cid, pl.ds(sid * ept, ept)],
                              e0_v, sem_in),
        pltpu.make_async_copy(
            pack_hbm.at[cid, pl.ds(n_edges + sid * ept, ept)], e1_v, sem_in),
        pltpu.make_async_copy(
            pack_hbm.at[cid, pl.ds(2 * n_edges + sid * ept, ept)], w_v,
            sem_in),
    ]
    for dsc in dins:
        dsc.start()

    @pl.loop(0, ROWS, unroll=4)
    def _(r):
        tab_v[r] = zeros

    @pl.loop(0, C // L)
    def _(r):
        idxa_v[pl.ds(r * L, L)] = iota + r * L

    idxb_v[...] = iota + C

    @pl.when(sid == 0)
    def _():
        pltpu.sync_copy(tab_v, shared)
    plsc.subcore_barrier()

    for dsc in dins:
        dsc.wait()

    @plsc.parallel_loop(0, ngrp, unroll=2)
    def _(g):
        j = g >> 3
        o = (g & 7) * L
        s16 = seg_v[pl.ds(g * L, L)]
        es = [emb_v[d, j, pl.ds(o, L)] for d in range(D)]
        for d in range(D):
            plsc.addupdate_scatter(tab_v, [s16, cols[d]], es[d])
        plsc.addupdate_scatter(tab_v, [C + (s16 >> 4), s16 & (L - 1)], ones)

    pltpu.sync_copy(tab_v.at[pl.ds(0, C)], shared.at[idxa_v], add=True)
    pltpu.sync_copy(tab_v.at[pl.ds(C, L)], shared.at[idxb_v], add=True)
    plsc.subcore_barrier()

    pltpu.sync_copy(shared, tab_v)

    @plsc.parallel_loop(0, C // L, carry=jnp.full((L,), -1, jnp.int32))
    def maxc(grp, mc):
        rows = iota + grp * L
        vs = [plsc.load_gather(tab_v, [rows, cols[d]]) for d in range(D)]
        nsq = _tree_sum([v * v for v in vs])
        rs = _rsqrt(jnp.maximum(nsq, 1e-20))
        for d in range(D):
            plsc.store_scatter(tab_v, [rows, cols[d]], vs[d] * rs)
        cnt = tab_v[C + grp]
        tab_v[C + L // 2 + grp] = 1.0 / cnt
        return jnp.maximum(mc, jnp.where(cnt > 0.0, rows, -1))

    @plsc.parallel_loop(0, ngrp, unroll=2, carry=zeros)
    def intra(g, acc):
        j = g >> 3
        o = (g & 7) * L
        s16 = seg_v[pl.ds(g * L, L)]
        es = [emb_v[d, j, pl.ds(o, L)] for d in range(D)]
        ms = [plsc.load_gather(tab_v, [s16, cols[d]]) for d in range(D)]
        inv16 = plsc.load_gather(tab_v,
                                 [C + L // 2 + (s16 >> 4), s16 & (L - 1)])
        dot = _tree_sum([e * m for e, m in zip(es, ms)])
        return acc + jnp.maximum((1.0 - DELTA_VAR) - dot, 0.0) * inv16

    inter = zeros
    for k in range(ept // L):
        a = plsc.bitcast(e0_v[pl.ds(k * L, L)], jnp.int32)
        b = plsc.bitcast(e1_v[pl.ds(k * L, L)], jnp.int32)
        mas = [plsc.load_gather(tab_v, [a, cols[d]]) for d in range(D)]
        mbs = [plsc.load_gather(tab_v, [b, cols[d]]) for d in range(D)]
        dd = _tree_sum([x * y for x, y in zip(mas, mbs)])
        wk = w_v[pl.ds(k * L, L)]
        inter = inter + jnp.maximum(DELTA_DIST - wk * (1.0 - dd), 0.0)

    c_div = jnp.broadcast_to(jnp.max(maxc) + 1, (L,)).astype(jnp.float32)

    inv_e = 1.0 / float(ept * NS)
    row_v[...] = BETA * (intra / c_div) + (ALPHA * inv_e) * inter
    pltpu.sync_copy(row_v, out_hbm.at[wid])


@jax.jit
def _run(emb, seg, pack):
    b, d, h, wdim = emb.shape
    pix = h * wdim // NS
    ept = pack.shape[1] // 3 // NS
    kern = pl.kernel(
        _body,
        out_type=jax.ShapeDtypeStruct((NC * NS, L), jnp.float32),
        mesh=plsc.VectorSubcoreMesh(core_axis_name="c", subcore_axis_name="s"),
        compiler_params=pltpu.CompilerParams(
            needs_layout_passes=False, use_tc_tiling_on_sc=False),
        scratch_types=[
            pltpu.VMEM((D, pix // 128, 128), jnp.float32),
            pltpu.VMEM((pix,), jnp.int32),
            pltpu.VMEM((ROWS, L), jnp.float32),
            pltpu.VMEM((ept,), jnp.float32),
            pltpu.VMEM((ept,), jnp.float32),
            pltpu.VMEM((ept,), jnp.float32),
            pltpu.VMEM((C,), jnp.int32),
            pltpu.VMEM((L,), jnp.int32),
            pltpu.VMEM((L,), jnp.float32),
            pltpu.VMEM_SHARED((ROWS, L), jnp.float32),
            pltpu.SemaphoreType.DMA,
        ],
    )
    out = kern(emb, seg, pack)
    return jnp.sum(out)


def kernel(embeddings, sp_seg, edges, weights, chunks=4):
    b = embeddings.shape[0]
    ef = jax.lax.bitcast_convert_type(edges.astype(jnp.int32), jnp.float32)
    pack = jnp.concatenate([ef[:, 0, :], ef[:, 1, :], weights], axis=1)
    return _run(embeddings, sp_seg.reshape(-1).astype(jnp.int32), pack)


# speedup vs baseline: 1.0072x; 1.0072x over previous
import jax
import jax.numpy as jnp
from jax import lax
from jax.experimental import pallas as pl
from jax.experimental.pallas import tpu as pltpu
from jax.experimental.pallas import tpu_sc as plsc

DELTA_VAR = 0.5
DELTA_DIST = 1.5
ALPHA = 1.0
BETA = 1.0

L = 16
NC = 2
NS = 16
D = 16
C = 128
ROWS = 144


def _rsqrt(x):
    i = plsc.bitcast(x, jnp.int32)
    i = 0x5F3759DF - (i >> 1)
    y = plsc.bitcast(i, jnp.float32)
    for _ in range(3):
        y = y * (1.5 - 0.5 * x * y * y)
    return y


def _tree_sum(xs):
    xs = list(xs)
    while len(xs) > 1:
        nxt = [xs[i] + xs[i + 1] for i in range(0, len(xs) - 1, 2)]
        if len(xs) % 2:
            nxt.append(xs[-1])
        xs = nxt
    return xs[0]


def _body(emb_hbm, seg_hbm, pack_hbm, out_hbm,
          emb_v, seg_v, tab_v, e0_v, e1_v, w_v, idxa_v, idxb_v,
          row_v, shared, sem_in):
    cid = lax.axis_index("c")
    sid = lax.axis_index("s")
    wid = cid * NS + sid

    pix = emb_v.shape[1] * emb_v.shape[2]
    ngrp = pix // L
    ept = e0_v.shape[0]

    iota = lax.iota(jnp.int32, L)
    zeros = jnp.zeros((L,), jnp.float32)
    ones = jnp.ones((L,), jnp.float32)
    cols = [jnp.full((L,), d, jnp.int32) for d in range(D)]

    rpt = pix // 128
    n_edges = ept * NS
    dins = [
        pltpu.make_async_copy(emb_hbm.at[cid, :, pl.ds(sid * rpt, rpt), :],
                              emb_v, sem_in),
        pltpu.make_async_copy(seg_hbm.at[cid, 0, pl.ds(sid * rpt, rpt), :],
                              seg_v, sem_in),
        pltpu.make_async_copy(pack_hbm.at[cid, pl.ds(sid * ept, ept)],
                              e0_v, sem_in),
        pltpu.make_async_copy(
            pack_hbm.at[cid, pl.ds(n_edges + sid * ept, ept)], e1_v, sem_in),
        pltpu.make_async_copy(
            pack_hbm.at[cid, pl.ds(2 * n_edges + sid * ept, ept)], w_v,
            sem_in),
    ]
    for dsc in dins:
        dsc.start()

    @pl.loop(0, ROWS, unroll=4)
    def _(r):
        tab_v[r] = zeros

    @pl.loop(0, C // L)
    def _(r):
        idxa_v[pl.ds(r * L, L)] = iota + r * L

    idxb_v[...] = iota + C

    @pl.when(sid == 0)
    def _():
        pltpu.sync_copy(tab_v, shared)
    plsc.subcore_barrier()

    for dsc in dins:
        dsc.wait()

    @plsc.parallel_loop(0, ngrp, unroll=2)
    def _(g):
        j = g >> 3
        o = (g & 7) * L
        s16 = seg_v[j, pl.ds(o, L)]
        es = [emb_v[d, j, pl.ds(o, L)] for d in range(D)]
        for d in range(D):
            plsc.addupdate_scatter(tab_v, [s16, cols[d]], es[d])
        plsc.addupdate_scatter(tab_v, [C + (s16 >> 4), s16 & (L - 1)], ones)

    dred = [
        pltpu.make_async_copy(tab_v.at[pl.ds(0, C)], shared.at[idxa_v],
                              sem_in),
        pltpu.make_async_copy(tab_v.at[pl.ds(C, L)], shared.at[idxb_v],
                              sem_in),
    ]
    for dsc in dred:
        dsc.start(add=True)
    for dsc in dred:
        dsc.wait()
    plsc.subcore_barrier()

    pltpu.sync_copy(shared, tab_v)

    @plsc.parallel_loop(0, C // L, carry=jnp.full((L,), -1, jnp.int32))
    def maxc(grp, mc):
        rows = iota + grp * L
        vs = [plsc.load_gather(tab_v, [rows, cols[d]]) for d in range(D)]
        nsq = _tree_sum([v * v for v in vs])
        rs = _rsqrt(jnp.maximum(nsq, 1e-20))
        for d in range(D):
            plsc.store_scatter(tab_v, [rows, cols[d]], vs[d] * rs)
        cnt = tab_v[C + grp]
        tab_v[C + L // 2 + grp] = 1.0 / cnt
        return jnp.maximum(mc, jnp.where(cnt > 0.0, rows, -1))

    @plsc.parallel_loop(0, ngrp, unroll=2, carry=zeros)
    def intra(g, acc):
        j = g >> 3
        o = (g & 7) * L
        s16 = seg_v[j, pl.ds(o, L)]
        es = [emb_v[d, j, pl.ds(o, L)] for d in range(D)]
        ms = [plsc.load_gather(tab_v, [s16, cols[d]]) for d in range(D)]
        inv16 = plsc.load_gather(tab_v,
                                 [C + L // 2 + (s16 >> 4), s16 & (L - 1)])
        dot = _tree_sum([e * m for e, m in zip(es, ms)])
        return acc + jnp.maximum((1.0 - DELTA_VAR) - dot, 0.0) * inv16

    inter = zeros
    for k in range(ept // L):
        a = plsc.bitcast(e0_v[pl.ds(k * L, L)], jnp.int32)
        b = plsc.bitcast(e1_v[pl.ds(k * L, L)], jnp.int32)
        mas = [plsc.load_gather(tab_v, [a, cols[d]]) for d in range(D)]
        mbs = [plsc.load_gather(tab_v, [b, cols[d]]) for d in range(D)]
        dd = _tree_sum([x * y for x, y in zip(mas, mbs)])
        wk = w_v[pl.ds(k * L, L)]
        inter = inter + jnp.maximum(DELTA_DIST - wk * (1.0 - dd), 0.0)

    c_div = jnp.broadcast_to(jnp.max(maxc) + 1, (L,)).astype(jnp.float32)

    inv_e = 1.0 / float(ept * NS)
    row_v[...] = BETA * (intra / c_div) + (ALPHA * inv_e) * inter
    pltpu.sync_copy(row_v, out_hbm.at[wid])


@jax.jit
def _run(emb, seg, pack):
    b, d, h, wdim = emb.shape
    pix = h * wdim // NS
    ept = pack.shape[1] // 3 // NS
    kern = pl.kernel(
        _body,
        out_type=jax.ShapeDtypeStruct((NC * NS, L), jnp.float32),
        mesh=plsc.VectorSubcoreMesh(core_axis_name="c", subcore_axis_name="s"),
        compiler_params=pltpu.CompilerParams(
            needs_layout_passes=False, use_tc_tiling_on_sc=False),
        scratch_types=[
            pltpu.VMEM((D, pix // 128, 128), jnp.float32),
            pltpu.VMEM((pix // 128, 128), jnp.int32),
            pltpu.VMEM((ROWS, L), jnp.float32),
            pltpu.VMEM((ept,), jnp.float32),
            pltpu.VMEM((ept,), jnp.float32),
            pltpu.VMEM((ept,), jnp.float32),
            pltpu.VMEM((C,), jnp.int32),
            pltpu.VMEM((L,), jnp.int32),
            pltpu.VMEM((L,), jnp.float32),
            pltpu.VMEM_SHARED((ROWS, L), jnp.float32),
            pltpu.SemaphoreType.DMA,
        ],
    )
    out = kern(emb, seg, pack)
    return jnp.sum(out)


def kernel(embeddings, sp_seg, edges, weights, chunks=4):
    b = embeddings.shape[0]
    ef = jax.lax.bitcast_convert_type(edges.astype(jnp.int32), jnp.float32)
    pack = jnp.concatenate([ef[:, 0, :], ef[:, 1, :], weights], axis=1)
    return _run(embeddings, sp_seg.astype(jnp.int32), pack)
